# baseline (speedup 1.0000x reference)
import functools

import jax
import jax.numpy as jnp
from jax.experimental import pallas as pl
from jax.experimental.pallas import tpu as pltpu

B, D, H, N = 16384, 64, 64, 64
GATES = 4 * H
BB = 128
BH = BB // 2


def _cell_kernel(x_ref, slots_ref, cum_ref, deltac_ref, delta_ref,
                 filled_ref, wqt_ref, wk_ref, wvt_ref, bv_ref, a12p_ref,
                 a3p_ref, w2p_ref, biasp_ref,
                 h_out_ref, slots_out_ref, cum_out_ref, delta_out_ref,
                 filled_out_ref):
    x = x_ref[...]
    slots = slots_ref[...]
    cum = cum_ref[...]
    deltac = deltac_ref[...]
    delta = delta_ref[...]
    filled = filled_ref[...]

    f32 = jnp.float32
    iota3 = jax.lax.broadcasted_iota(jnp.int32, (BB, N, 1), 1)
    iota13 = jax.lax.broadcasted_iota(jnp.int32, (BB, 1, N), 2)

    q = jnp.dot(x, wqt_ref[...], preferred_element_type=f32)
    qk = jnp.dot(q, wk_ref[...], preferred_element_type=f32)
    sims = jnp.sum(slots * qk[:, None, :], axis=2, keepdims=True)

    empty = filled == 0.0
    idx_empty = jnp.min(jnp.where(empty, iota13, N), axis=2, keepdims=True)
    sims_max = jnp.max(sims, axis=1, keepdims=True)
    idx_cont = jnp.min(jnp.where(sims == sims_max, iota3, N),
                       axis=1, keepdims=True)
    idx = jnp.where(idx_empty < N, idx_empty, idx_cont)
    onehot = iota3 == idx
    onehot13 = iota13 == idx

    v = jnp.dot(x, wvt_ref[...], preferred_element_type=f32) + bv_ref[...]
    deltac_new = jnp.where(onehot, 0.0, deltac + 1.0)
    cum_new = jnp.where(onehot, x[:, None, :], cum + x[:, None, :])
    slots_new = jnp.where(onehot, v[:, None, :], slots)

    slots_out_ref[...] = slots_new
    cum_out_ref[...] = cum_new
    delta_out_ref[...] = jnp.where(onehot13, 0.0, delta + 1.0)
    filled_out_ref[...] = jnp.where(onehot13, 1.0, filled)

    lane = jax.lax.broadcasted_iota(jnp.int32, (1, 2 * GATES), 1)
    gsel = jnp.logical_and(lane >= 4 * H, lane < 6 * H)
    gscale = jnp.where(gsel, 1.0, 0.5)
    act_sc = jnp.where(gsel, 1.0, 0.5)
    act_off = jnp.where(gsel, 0.0, 0.5)
    a12p = a12p_ref[...] * gscale
    a3p = a3p_ref[...] * gscale
    w2p = (w2p_ref[...] * gscale).astype(jnp.bfloat16)
    biasp = biasp_ref[...] * gscale
    lane_a = (lane % (2 * H)) < H
    a3p_a = jnp.where(lane_a, a3p, 0.0)
    a3p_b = jnp.where(lane_a, 0.0, a3p)

    x2 = jnp.concatenate(
        [slots_new.reshape(BB * N, D), cum_new.reshape(BB * N, D)], axis=1)
    x2p = jnp.concatenate([x2[:BH * N], x2[BH * N:]], axis=1)
    ginp = jnp.dot(x2p.astype(jnp.bfloat16), a12p.astype(jnp.bfloat16),
                   preferred_element_type=f32)
    ginp3 = (ginp.reshape(BH, N, 2 * GATES)
             + deltac_new[:BH] * a3p_a[None]
             + deltac_new[BH:] * a3p_b[None]
             + biasp[None])
    gin_t = jnp.transpose(ginp3, (1, 0, 2))

    hp = jnp.zeros((BH, 2 * H), dtype=f32)
    cp = jnp.zeros((BH, 2 * H), dtype=f32)
    for t in range(N):
        g = gin_t[t] + jnp.dot(hp.astype(jnp.bfloat16), w2p,
                               preferred_element_type=f32)
        th = jnp.tanh(g)
        act = th * act_sc + act_off
        i_p = act[:, 0:2 * H]
        f_p = act[:, 2 * H:4 * H]
        g_p = act[:, 4 * H:6 * H]
        o_p = act[:, 6 * H:8 * H]
        cp = f_p * cp + i_p * g_p
        hp = o_p * jnp.tanh(cp)
    h_out_ref[:BH, :] = hp[:, 0:H]
    h_out_ref[BH:, :] = hp[:, H:2 * H]


@functools.partial(jax.jit, static_argnames=("interpret",))
def _run(x_t, slots, cum_feats, deltac3, delta13, filled13,
         wqt, wk, wvt, bv, a12p, a3p, w2p, biasp, interpret=False):
    grid = (B // BB,)
    row = lambda i: (i, 0)
    row3 = lambda i: (i, 0, 0)
    rep = lambda i: (0, 0)
    in_specs = [
        pl.BlockSpec((BB, D), row),
        pl.BlockSpec((BB, N, D), row3),
        pl.BlockSpec((BB, N, D), row3),
        pl.BlockSpec((BB, N, 1), row3),
        pl.BlockSpec((BB, 1, N), row3),
        pl.BlockSpec((BB, 1, N), row3),
        pl.BlockSpec((D, H), rep),
        pl.BlockSpec((H, D), rep),
        pl.BlockSpec((D, D), rep),
        pl.BlockSpec((1, D), rep),
        pl.BlockSpec((4 * D, 2 * GATES), rep),
        pl.BlockSpec((1, 2 * GATES), rep),
        pl.BlockSpec((2 * H, 2 * GATES), rep),
        pl.BlockSpec((1, 2 * GATES), rep),
    ]
    out_specs = [
        pl.BlockSpec((BB, H), row),
        pl.BlockSpec((BB, N, D), row3),
        pl.BlockSpec((BB, N, D), row3),
        pl.BlockSpec((BB, 1, N), row3),
        pl.BlockSpec((BB, 1, N), row3),
    ]
    out_shapes = [
        jax.ShapeDtypeStruct((B, H), jnp.float32),
        jax.ShapeDtypeStruct((B, N, D), jnp.float32),
        jax.ShapeDtypeStruct((B, N, D), jnp.float32),
        jax.ShapeDtypeStruct((B, 1, N), jnp.float32),
        jax.ShapeDtypeStruct((B, 1, N), jnp.float32),
    ]
    return pl.pallas_call(
        _cell_kernel,
        grid=grid,
        in_specs=in_specs,
        out_specs=out_specs,
        out_shape=out_shapes,
        compiler_params=pltpu.CompilerParams(
            dimension_semantics=("arbitrary",)),
        interpret=interpret,
    )(x_t, slots, cum_feats, deltac3, delta13, filled13,
      wqt, wk, wvt, bv, a12p, a3p, w2p, biasp)


def _pack_cols(w):
    r = w.shape[0]
    wg = w.reshape(r, 4, H)
    z = jnp.zeros_like(wg)
    return jnp.concatenate([wg, z], axis=2).reshape(r, 2 * GATES)


def _pack_cols_b(w):
    r = w.shape[0]
    wg = w.reshape(r, 4, H)
    z = jnp.zeros_like(wg)
    return jnp.concatenate([z, wg], axis=2).reshape(r, 2 * GATES)


def kernel(x_t, h_mem_prev, slots, cum_feats, delta_t, filled,
           W_q, W_k, W_v, b_v, W_ih, W_hh, b_ih, b_hh):
    del h_mem_prev
    deltac3 = delta_t.reshape(B, N, 1)
    delta13 = delta_t.reshape(B, 1, N)
    filled13 = filled.astype(jnp.float32).reshape(B, 1, N)
    wqt = W_q.T
    wvt = W_v.T
    bv = b_v.reshape(1, D)
    a1 = W_ih[:, :D].T
    a2 = W_ih[:, D:2 * D].T
    a3 = W_ih[:, 2 * D].reshape(1, GATES)
    whht = W_hh.T
    bias = (b_ih + b_hh).reshape(1, GATES)
    a12p = jnp.concatenate([
        _pack_cols(a1), _pack_cols(a2),
        _pack_cols_b(a1), _pack_cols_b(a2)], axis=0)
    w2p = jnp.concatenate([
        _pack_cols(whht), _pack_cols_b(whht)], axis=0)
    a3p = _pack_cols(a3) + _pack_cols_b(a3)
    biasp = _pack_cols(bias) + _pack_cols_b(bias)
    h_mem, slots_o, cum_o, delta_o, filled_o = _run(
        x_t, slots, cum_feats, deltac3, delta13, filled13,
        wqt, W_k, wvt, bv, a12p, a3p, w2p, biasp)
    return (h_mem, slots_o, cum_o, delta_o.reshape(B, N),
            filled_o.reshape(B, N) > 0.5)


# speedup vs baseline: 3.4914x; 3.4914x over previous
import functools

import jax
import jax.numpy as jnp
from jax.experimental import pallas as pl
from jax.experimental.pallas import tpu as pltpu

B, D, H, N = 16384, 64, 64, 64
GATES = 4 * H
BBL = 256


def _cell_kernel(x_ref, slots_ref, cum_ref, delta_ref, filled_ref,
                 wq_ref, wkt_ref, wv_ref, bv_ref, a12_ref, a3_ref,
                 whh_ref, bias_ref,
                 h_out_ref, slots_out_ref, cum_out_ref, delta_out_ref,
                 filled_out_ref):
    x = x_ref[...]
    slots = slots_ref[...]
    cum = cum_ref[...]
    delta = delta_ref[...]
    filled = filled_ref[...]

    f32 = jnp.float32
    iota_n = jax.lax.broadcasted_iota(jnp.int32, (N, BBL), 0)

    q = jnp.dot(wq_ref[...], x, preferred_element_type=f32)
    qk = jnp.dot(wkt_ref[...], q, preferred_element_type=f32)
    sims = jnp.sum(slots * qk[None], axis=1)

    empty = filled == 0.0
    idx_empty = jnp.min(jnp.where(empty, iota_n, N), axis=0, keepdims=True)
    sims_max = jnp.max(sims, axis=0, keepdims=True)
    idx_cont = jnp.min(jnp.where(sims == sims_max, iota_n, N),
                       axis=0, keepdims=True)
    idx = jnp.where(idx_empty < N, idx_empty, idx_cont)
    onehot = iota_n == idx
    oh3 = onehot[:, None, :]

    v = jnp.dot(wv_ref[...], x, preferred_element_type=f32) + bv_ref[...]
    delta_new = jnp.where(onehot, 0.0, delta + 1.0)
    cum_new = jnp.where(oh3, x[None], cum + x[None])
    slots_new = jnp.where(oh3, v[None], slots)

    slots_out_ref[...] = slots_new
    cum_out_ref[...] = cum_new
    delta_out_ref[...] = delta_new
    filled_out_ref[...] = jnp.where(onehot, 1.0, filled)

    grow = jax.lax.broadcasted_iota(jnp.int32, (GATES, 1), 0)
    gsel = jnp.logical_and(grow >= 2 * H, grow < 3 * H)
    gscale = jnp.where(gsel, 1.0, 0.5)
    a12 = a12_ref[...] * gscale
    a3 = a3_ref[...] * gscale
    whh = whh_ref[...] * gscale
    bias = bias_ref[...] * gscale

    h = jnp.zeros((H, BBL), dtype=f32)
    c = jnp.zeros((H, BBL), dtype=f32)
    for t in range(N):
        xcat = jnp.concatenate([slots_new[t], cum_new[t]], axis=0)
        g = (jnp.dot(a12, xcat, preferred_element_type=f32)
             + jnp.dot(whh, h, preferred_element_type=f32)
             + a3 * delta_new[t][None]
             + bias)
        act = jnp.tanh(g)
        i_g = 0.5 * act[0:H] + 0.5
        f_g = 0.5 * act[H:2 * H] + 0.5
        g_g = act[2 * H:3 * H]
        o_g = 0.5 * act[3 * H:4 * H] + 0.5
        c = f_g * c + i_g * g_g
        h = o_g * jnp.tanh(c)
    h_out_ref[...] = h


@functools.partial(jax.jit, static_argnames=("interpret",))
def _run(xt, slots_t, cum_t, delta_t2, filled_t2,
         wq, wkt, wv, bv, a12, a3, whh, bias, interpret=False):
    grid = (B // BBL,)
    lane2 = lambda i: (0, i)
    lane3 = lambda i: (0, 0, i)
    rep = lambda i: (0, 0)
    in_specs = [
        pl.BlockSpec((D, BBL), lane2),
        pl.BlockSpec((N, D, BBL), lane3),
        pl.BlockSpec((N, D, BBL), lane3),
        pl.BlockSpec((N, BBL), lane2),
        pl.BlockSpec((N, BBL), lane2),
        pl.BlockSpec((H, D), rep),
        pl.BlockSpec((D, H), rep),
        pl.BlockSpec((D, D), rep),
        pl.BlockSpec((D, 1), rep),
        pl.BlockSpec((GATES, 2 * D), rep),
        pl.BlockSpec((GATES, 1), rep),
        pl.BlockSpec((GATES, H), rep),
        pl.BlockSpec((GATES, 1), rep),
    ]
    out_specs = [
        pl.BlockSpec((H, BBL), lane2),
        pl.BlockSpec((N, D, BBL), lane3),
        pl.BlockSpec((N, D, BBL), lane3),
        pl.BlockSpec((N, BBL), lane2),
        pl.BlockSpec((N, BBL), lane2),
    ]
    out_shapes = [
        jax.ShapeDtypeStruct((H, B), jnp.float32),
        jax.ShapeDtypeStruct((N, D, B), jnp.float32),
        jax.ShapeDtypeStruct((N, D, B), jnp.float32),
        jax.ShapeDtypeStruct((N, B), jnp.float32),
        jax.ShapeDtypeStruct((N, B), jnp.float32),
    ]
    return pl.pallas_call(
        _cell_kernel,
        grid=grid,
        in_specs=in_specs,
        out_specs=out_specs,
        out_shape=out_shapes,
        compiler_params=pltpu.CompilerParams(
            dimension_semantics=("arbitrary",)),
        interpret=interpret,
    )(xt, slots_t, cum_t, delta_t2, filled_t2,
      wq, wkt, wv, bv, a12, a3, whh, bias)


def kernel(x_t, h_mem_prev, slots, cum_feats, delta_t, filled,
           W_q, W_k, W_v, b_v, W_ih, W_hh, b_ih, b_hh):
    del h_mem_prev
    xt = x_t.T
    slots_t = jnp.transpose(slots, (1, 2, 0))
    cum_t = jnp.transpose(cum_feats, (1, 2, 0))
    delta_t2 = delta_t.T
    filled_t2 = filled.T.astype(jnp.float32)
    bv = b_v.reshape(D, 1)
    a12 = W_ih[:, :2 * D]
    a3 = W_ih[:, 2 * D].reshape(GATES, 1)
    bias = (b_ih + b_hh).reshape(GATES, 1)
    h_mem, slots_o, cum_o, delta_o, filled_o = _run(
        xt, slots_t, cum_t, delta_t2, filled_t2,
        W_q, W_k.T, W_v, bv, a12, a3, W_hh, bias)
    return (h_mem.T, jnp.transpose(slots_o, (2, 0, 1)),
            jnp.transpose(cum_o, (2, 0, 1)), delta_o.T,
            filled_o.T > 0.5)


# speedup vs baseline: 3.4923x; 1.0003x over previous
import functools

import jax
import jax.numpy as jnp
from jax.experimental import pallas as pl
from jax.experimental.pallas import tpu as pltpu

B, D, H, N = 16384, 64, 64, 64
GATES = 4 * H
BBL = 256


def _cell_kernel(x_ref, slots_ref, cum_ref, delta_ref, filled_ref,
                 wq_ref, wkt_ref, wv_ref, bv_ref, a12_ref, a3_ref,
                 whh_ref, bias_ref,
                 h_out_ref, slots_out_ref, cum_out_ref, delta_out_ref,
                 filled_out_ref):
    x = x_ref[...]
    slots = slots_ref[...]
    cum = cum_ref[...]
    delta = delta_ref[...]
    filled = filled_ref[...]

    f32 = jnp.float32
    iota_n = jax.lax.broadcasted_iota(jnp.int32, (N, BBL), 0)

    q = jnp.dot(wq_ref[...], x, preferred_element_type=f32)
    qk = jnp.dot(wkt_ref[...], q, preferred_element_type=f32)
    sims = jnp.sum(slots * qk[None], axis=1)

    empty = filled == 0.0
    idx_empty = jnp.min(jnp.where(empty, iota_n, N), axis=0, keepdims=True)
    sims_max = jnp.max(sims, axis=0, keepdims=True)
    idx_cont = jnp.min(jnp.where(sims == sims_max, iota_n, N),
                       axis=0, keepdims=True)
    idx = jnp.where(idx_empty < N, idx_empty, idx_cont)
    onehot = iota_n == idx
    oh3 = onehot[:, None, :]

    v = jnp.dot(wv_ref[...], x, preferred_element_type=f32) + bv_ref[...]
    delta_new = jnp.where(onehot, 0.0, delta + 1.0)
    cum_new = jnp.where(oh3, x[None], cum + x[None])
    slots_new = jnp.where(oh3, v[None], slots)

    slots_out_ref[...] = slots_new
    cum_out_ref[...] = cum_new
    delta_out_ref[...] = delta_new
    filled_out_ref[...] = jnp.where(onehot, 1.0, filled)

    grow = jax.lax.broadcasted_iota(jnp.int32, (GATES, 1), 0)
    gsel = jnp.logical_and(grow >= 2 * H, grow < 3 * H)
    gscale = jnp.where(gsel, 1.0, 0.5)
    a12 = (a12_ref[...] * gscale).astype(jnp.bfloat16)
    a3 = a3_ref[...] * gscale
    whh = (whh_ref[...] * gscale).astype(jnp.bfloat16)
    bias = bias_ref[...] * gscale

    h = jnp.zeros((H, BBL), dtype=f32)
    c = jnp.zeros((H, BBL), dtype=f32)
    for t in range(N):
        xcat = jnp.concatenate([slots_new[t], cum_new[t]],
                               axis=0).astype(jnp.bfloat16)
        g = (jnp.dot(a12, xcat, preferred_element_type=f32)
             + jnp.dot(whh, h.astype(jnp.bfloat16),
                       preferred_element_type=f32)
             + a3 * delta_new[t][None]
             + bias)
        act = jnp.tanh(g)
        i_g = 0.5 * act[0:H] + 0.5
        f_g = 0.5 * act[H:2 * H] + 0.5
        g_g = act[2 * H:3 * H]
        o_g = 0.5 * act[3 * H:4 * H] + 0.5
        c = f_g * c + i_g * g_g
        h = o_g * jnp.tanh(c)
    h_out_ref[...] = h


@functools.partial(jax.jit, static_argnames=("interpret",))
def _run(xt, slots_t, cum_t, delta_t2, filled_t2,
         wq, wkt, wv, bv, a12, a3, whh, bias, interpret=False):
    grid = (B // BBL,)
    lane2 = lambda i: (0, i)
    lane3 = lambda i: (0, 0, i)
    rep = lambda i: (0, 0)
    in_specs = [
        pl.BlockSpec((D, BBL), lane2),
        pl.BlockSpec((N, D, BBL), lane3),
        pl.BlockSpec((N, D, BBL), lane3),
        pl.BlockSpec((N, BBL), lane2),
        pl.BlockSpec((N, BBL), lane2),
        pl.BlockSpec((H, D), rep),
        pl.BlockSpec((D, H), rep),
        pl.BlockSpec((D, D), rep),
        pl.BlockSpec((D, 1), rep),
        pl.BlockSpec((GATES, 2 * D), rep),
        pl.BlockSpec((GATES, 1), rep),
        pl.BlockSpec((GATES, H), rep),
        pl.BlockSpec((GATES, 1), rep),
    ]
    out_specs = [
        pl.BlockSpec((H, BBL), lane2),
        pl.BlockSpec((N, D, BBL), lane3),
        pl.BlockSpec((N, D, BBL), lane3),
        pl.BlockSpec((N, BBL), lane2),
        pl.BlockSpec((N, BBL), lane2),
    ]
    out_shapes = [
        jax.ShapeDtypeStruct((H, B), jnp.float32),
        jax.ShapeDtypeStruct((N, D, B), jnp.float32),
        jax.ShapeDtypeStruct((N, D, B), jnp.float32),
        jax.ShapeDtypeStruct((N, B), jnp.float32),
        jax.ShapeDtypeStruct((N, B), jnp.float32),
    ]
    return pl.pallas_call(
        _cell_kernel,
        grid=grid,
        in_specs=in_specs,
        out_specs=out_specs,
        out_shape=out_shapes,
        compiler_params=pltpu.CompilerParams(
            dimension_semantics=("arbitrary",)),
        interpret=interpret,
    )(xt, slots_t, cum_t, delta_t2, filled_t2,
      wq, wkt, wv, bv, a12, a3, whh, bias)


def kernel(x_t, h_mem_prev, slots, cum_feats, delta_t, filled,
           W_q, W_k, W_v, b_v, W_ih, W_hh, b_ih, b_hh):
    del h_mem_prev
    xt = x_t.T
    slots_t = jnp.transpose(slots, (1, 2, 0))
    cum_t = jnp.transpose(cum_feats, (1, 2, 0))
    delta_t2 = delta_t.T
    filled_t2 = filled.T.astype(jnp.float32)
    bv = b_v.reshape(D, 1)
    a12 = W_ih[:, :2 * D]
    a3 = W_ih[:, 2 * D].reshape(GATES, 1)
    bias = (b_ih + b_hh).reshape(GATES, 1)
    h_mem, slots_o, cum_o, delta_o, filled_o = _run(
        xt, slots_t, cum_t, delta_t2, filled_t2,
        W_q, W_k.T, W_v, bv, a12, a3, W_hh, bias)
    return (h_mem.T, jnp.transpose(slots_o, (2, 0, 1)),
            jnp.transpose(cum_o, (2, 0, 1)), delta_o.T,
            filled_o.T > 0.5)


# speedup vs baseline: 4.2896x; 1.2283x over previous
import functools

import jax
import jax.numpy as jnp
from jax.experimental import pallas as pl
from jax.experimental.pallas import tpu as pltpu

B, D, H, N = 16384, 64, 64, 64
GATES = 4 * H
BBL = 256


def _cell_kernel(x_ref, slots_ref, cum_ref, delta_ref, filled_ref,
                 wq_ref, wkt_ref, wv_ref, bv_ref, a12_ref, a3_ref,
                 whh_ref, bias_ref,
                 h_out_ref, slots_out_ref, cum_out_ref, delta_out_ref,
                 filled_out_ref):
    x = x_ref[...]
    slots = slots_ref[...]
    cum = cum_ref[...]
    delta = delta_ref[...]
    filled = filled_ref[...]

    f32 = jnp.float32
    iota_n = jax.lax.broadcasted_iota(jnp.int32, (N, BBL), 0)

    q = jnp.dot(wq_ref[...], x, preferred_element_type=f32)
    qk = jnp.dot(wkt_ref[...], q, preferred_element_type=f32)
    sims = jnp.sum(slots * qk[None], axis=1)

    empty = filled == 0.0
    idx_empty = jnp.min(jnp.where(empty, iota_n, N), axis=0, keepdims=True)
    sims_max = jnp.max(sims, axis=0, keepdims=True)
    idx_cont = jnp.min(jnp.where(sims == sims_max, iota_n, N),
                       axis=0, keepdims=True)
    idx = jnp.where(idx_empty < N, idx_empty, idx_cont)
    onehot = iota_n == idx
    oh3 = onehot[:, None, :]

    v = jnp.dot(wv_ref[...], x, preferred_element_type=f32) + bv_ref[...]
    delta_new = jnp.where(onehot, 0.0, delta + 1.0)
    cum_new = jnp.where(oh3, x[None], cum + x[None])
    slots_new = jnp.where(oh3, v[None], slots)

    slots_out_ref[...] = slots_new
    cum_out_ref[...] = cum_new
    delta_out_ref[...] = delta_new
    filled_out_ref[...] = jnp.where(onehot, 1.0, filled)

    grow = jax.lax.broadcasted_iota(jnp.int32, (GATES, 1), 0)
    gsel = jnp.logical_and(grow >= 2 * H, grow < 3 * H)
    gscale = jnp.where(gsel, 1.0, 0.5)
    a12 = a12_ref[...] * gscale
    a1 = a12[:, 0:D]
    a2 = a12[:, D:2 * D]
    a3 = a3_ref[...] * gscale
    whh = whh_ref[...] * gscale
    bias = bias_ref[...] * gscale

    LH = BBL // 2
    hs = [jnp.zeros((H, LH), dtype=f32) for _ in range(2)]
    cs = [jnp.zeros((H, LH), dtype=f32) for _ in range(2)]
    for t in range(N):
        st = slots_new[t]
        ct = cum_new[t]
        dt = delta_new[t]
        for k in range(2):
            sl = slice(k * LH, (k + 1) * LH)
            g = (jnp.dot(a1, st[:, sl], preferred_element_type=f32)
                 + jnp.dot(a2, ct[:, sl], preferred_element_type=f32)
                 + jnp.dot(whh, hs[k], preferred_element_type=f32)
                 + a3 * dt[None, sl]
                 + bias)
            act = jnp.tanh(g)
            i_g = 0.5 * act[0:H] + 0.5
            f_g = 0.5 * act[H:2 * H] + 0.5
            g_g = act[2 * H:3 * H]
            o_g = 0.5 * act[3 * H:4 * H] + 0.5
            cs[k] = f_g * cs[k] + i_g * g_g
            hs[k] = o_g * jnp.tanh(cs[k])
    h_out_ref[:, 0:LH] = hs[0]
    h_out_ref[:, LH:BBL] = hs[1]


@functools.partial(jax.jit, static_argnames=("interpret",))
def _run(xt, slots_t, cum_t, delta_t2, filled_t2,
         wq, wkt, wv, bv, a12, a3, whh, bias, interpret=False):
    grid = (B // BBL,)
    lane2 = lambda i: (0, i)
    lane3 = lambda i: (0, 0, i)
    rep = lambda i: (0, 0)
    in_specs = [
        pl.BlockSpec((D, BBL), lane2),
        pl.BlockSpec((N, D, BBL), lane3),
        pl.BlockSpec((N, D, BBL), lane3),
        pl.BlockSpec((N, BBL), lane2),
        pl.BlockSpec((N, BBL), lane2),
        pl.BlockSpec((H, D), rep),
        pl.BlockSpec((D, H), rep),
        pl.BlockSpec((D, D), rep),
        pl.BlockSpec((D, 1), rep),
        pl.BlockSpec((GATES, 2 * D), rep),
        pl.BlockSpec((GATES, 1), rep),
        pl.BlockSpec((GATES, H), rep),
        pl.BlockSpec((GATES, 1), rep),
    ]
    out_specs = [
        pl.BlockSpec((H, BBL), lane2),
        pl.BlockSpec((N, D, BBL), lane3),
        pl.BlockSpec((N, D, BBL), lane3),
        pl.BlockSpec((N, BBL), lane2),
        pl.BlockSpec((N, BBL), lane2),
    ]
    out_shapes = [
        jax.ShapeDtypeStruct((H, B), jnp.float32),
        jax.ShapeDtypeStruct((N, D, B), jnp.float32),
        jax.ShapeDtypeStruct((N, D, B), jnp.float32),
        jax.ShapeDtypeStruct((N, B), jnp.float32),
        jax.ShapeDtypeStruct((N, B), jnp.float32),
    ]
    return pl.pallas_call(
        _cell_kernel,
        grid=grid,
        in_specs=in_specs,
        out_specs=out_specs,
        out_shape=out_shapes,
        compiler_params=pltpu.CompilerParams(
            dimension_semantics=("arbitrary",)),
        interpret=interpret,
    )(xt, slots_t, cum_t, delta_t2, filled_t2,
      wq, wkt, wv, bv, a12, a3, whh, bias)


def kernel(x_t, h_mem_prev, slots, cum_feats, delta_t, filled,
           W_q, W_k, W_v, b_v, W_ih, W_hh, b_ih, b_hh):
    del h_mem_prev
    xt = x_t.T
    slots_t = jnp.transpose(slots, (1, 2, 0))
    cum_t = jnp.transpose(cum_feats, (1, 2, 0))
    delta_t2 = delta_t.T
    filled_t2 = filled.T.astype(jnp.float32)
    bv = b_v.reshape(D, 1)
    a12 = W_ih[:, :2 * D]
    a3 = W_ih[:, 2 * D].reshape(GATES, 1)
    bias = (b_ih + b_hh).reshape(GATES, 1)
    h_mem, slots_o, cum_o, delta_o, filled_o = _run(
        xt, slots_t, cum_t, delta_t2, filled_t2,
        W_q, W_k.T, W_v, bv, a12, a3, W_hh, bias)
    return (h_mem.T, jnp.transpose(slots_o, (2, 0, 1)),
            jnp.transpose(cum_o, (2, 0, 1)), delta_o.T,
            filled_o.T > 0.5)


# speedup vs baseline: 4.3625x; 1.0170x over previous
import functools

import jax
import jax.numpy as jnp
from jax.experimental import pallas as pl
from jax.experimental.pallas import tpu as pltpu

B, D, H, N = 16384, 64, 64, 64
GATES = 4 * H
BBL = 256


def _cell_kernel(x_ref, slots_ref, cum_ref, delta_ref, filled_ref,
                 wq_ref, wkt_ref, wv_ref, bv_ref, a12_ref, a3_ref,
                 whh_ref, bias_ref,
                 h_out_ref, slots_out_ref, cum_out_ref, delta_out_ref,
                 filled_out_ref):
    x = x_ref[...]
    slots = slots_ref[...]
    cum = cum_ref[...]
    delta = delta_ref[...]
    filled = filled_ref[...]

    f32 = jnp.float32
    iota_n = jax.lax.broadcasted_iota(jnp.int32, (N, BBL), 0)

    q = jnp.dot(wq_ref[...], x, preferred_element_type=f32)
    qk = jnp.dot(wkt_ref[...], q, preferred_element_type=f32)
    sims = jnp.sum(slots * qk[None], axis=1)

    empty = filled == 0.0
    idx_empty = jnp.min(jnp.where(empty, iota_n, N), axis=0, keepdims=True)
    sims_max = jnp.max(sims, axis=0, keepdims=True)
    idx_cont = jnp.min(jnp.where(sims == sims_max, iota_n, N),
                       axis=0, keepdims=True)
    idx = jnp.where(idx_empty < N, idx_empty, idx_cont)
    onehot = iota_n == idx
    oh3 = onehot[:, None, :]

    v = jnp.dot(wv_ref[...], x, preferred_element_type=f32) + bv_ref[...]
    delta_new = jnp.where(onehot, 0.0, delta + 1.0)
    cum_new = jnp.where(oh3, x[None], cum + x[None])
    slots_new = jnp.where(oh3, v[None], slots)

    slots_out_ref[...] = slots_new
    cum_out_ref[...] = cum_new
    delta_out_ref[...] = delta_new
    filled_out_ref[...] = jnp.where(onehot, 1.0, filled)

    grow = jax.lax.broadcasted_iota(jnp.int32, (GATES, 1), 0)
    gsel = jnp.logical_and(grow >= 2 * H, grow < 3 * H)
    gscale = jnp.where(gsel, 1.0, 0.5)
    bf16 = jnp.bfloat16
    a12 = a12_ref[...] * gscale
    a1 = a12[:, 0:D].astype(bf16)
    a2 = a12[:, D:2 * D].astype(bf16)
    a3 = a3_ref[...] * gscale
    whh = (whh_ref[...] * gscale).astype(bf16)
    bias = bias_ref[...] * gscale
    slots_b = slots_new.astype(bf16)
    cum_b = cum_new.astype(bf16)

    LH = BBL // 2
    hs = [jnp.zeros((H, LH), dtype=f32) for _ in range(2)]
    cs = [jnp.zeros((H, LH), dtype=f32) for _ in range(2)]
    for t in range(N):
        st = slots_b[t]
        ct = cum_b[t]
        dt = delta_new[t]
        for k in range(2):
            sl = slice(k * LH, (k + 1) * LH)
            g = (jnp.dot(a1, st[:, sl], preferred_element_type=f32)
                 + jnp.dot(a2, ct[:, sl], preferred_element_type=f32)
                 + jnp.dot(whh, hs[k].astype(bf16),
                           preferred_element_type=f32)
                 + a3 * dt[None, sl]
                 + bias)
            act = jnp.tanh(g)
            i_g = 0.5 * act[0:H] + 0.5
            f_g = 0.5 * act[H:2 * H] + 0.5
            g_g = act[2 * H:3 * H]
            o_g = 0.5 * act[3 * H:4 * H] + 0.5
            cs[k] = f_g * cs[k] + i_g * g_g
            hs[k] = o_g * jnp.tanh(cs[k])
    h_out_ref[:, 0:LH] = hs[0]
    h_out_ref[:, LH:BBL] = hs[1]


@functools.partial(jax.jit, static_argnames=("interpret",))
def _run(xt, slots_t, cum_t, delta_t2, filled_t2,
         wq, wkt, wv, bv, a12, a3, whh, bias, interpret=False):
    grid = (B // BBL,)
    lane2 = lambda i: (0, i)
    lane3 = lambda i: (0, 0, i)
    rep = lambda i: (0, 0)
    in_specs = [
        pl.BlockSpec((D, BBL), lane2),
        pl.BlockSpec((N, D, BBL), lane3),
        pl.BlockSpec((N, D, BBL), lane3),
        pl.BlockSpec((N, BBL), lane2),
        pl.BlockSpec((N, BBL), lane2),
        pl.BlockSpec((H, D), rep),
        pl.BlockSpec((D, H), rep),
        pl.BlockSpec((D, D), rep),
        pl.BlockSpec((D, 1), rep),
        pl.BlockSpec((GATES, 2 * D), rep),
        pl.BlockSpec((GATES, 1), rep),
        pl.BlockSpec((GATES, H), rep),
        pl.BlockSpec((GATES, 1), rep),
    ]
    out_specs = [
        pl.BlockSpec((H, BBL), lane2),
        pl.BlockSpec((N, D, BBL), lane3),
        pl.BlockSpec((N, D, BBL), lane3),
        pl.BlockSpec((N, BBL), lane2),
        pl.BlockSpec((N, BBL), lane2),
    ]
    out_shapes = [
        jax.ShapeDtypeStruct((H, B), jnp.float32),
        jax.ShapeDtypeStruct((N, D, B), jnp.float32),
        jax.ShapeDtypeStruct((N, D, B), jnp.float32),
        jax.ShapeDtypeStruct((N, B), jnp.float32),
        jax.ShapeDtypeStruct((N, B), jnp.float32),
    ]
    return pl.pallas_call(
        _cell_kernel,
        grid=grid,
        in_specs=in_specs,
        out_specs=out_specs,
        out_shape=out_shapes,
        compiler_params=pltpu.CompilerParams(
            dimension_semantics=("arbitrary",)),
        interpret=interpret,
    )(xt, slots_t, cum_t, delta_t2, filled_t2,
      wq, wkt, wv, bv, a12, a3, whh, bias)


def kernel(x_t, h_mem_prev, slots, cum_feats, delta_t, filled,
           W_q, W_k, W_v, b_v, W_ih, W_hh, b_ih, b_hh):
    del h_mem_prev
    xt = x_t.T
    slots_t = jnp.transpose(slots, (1, 2, 0))
    cum_t = jnp.transpose(cum_feats, (1, 2, 0))
    delta_t2 = delta_t.T
    filled_t2 = filled.T.astype(jnp.float32)
    bv = b_v.reshape(D, 1)
    a12 = W_ih[:, :2 * D]
    a3 = W_ih[:, 2 * D].reshape(GATES, 1)
    bias = (b_ih + b_hh).reshape(GATES, 1)
    h_mem, slots_o, cum_o, delta_o, filled_o = _run(
        xt, slots_t, cum_t, delta_t2, filled_t2,
        W_q, W_k.T, W_v, bv, a12, a3, W_hh, bias)
    return (h_mem.T, jnp.transpose(slots_o, (2, 0, 1)),
            jnp.transpose(cum_o, (2, 0, 1)), delta_o.T,
            filled_o.T > 0.5)
